# Initial kernel scaffold; baseline (speedup 1.0000x reference)
import jax
import jax.numpy as jnp
from jax.experimental import pallas as pl


def kernel(pred, observed, std, L_table):
    raise NotImplementedError("write your pallas kernel here")


# speedup vs baseline: 1.4141x; 1.4141x over previous
import functools

import jax
import jax.numpy as jnp
from jax import lax
from jax.experimental import pallas as pl
from jax.experimental.pallas import tpu as pltpu
from jax.experimental.pallas import tpu_sc as plsc

N_PMTS = 494
BATCH = 16384
M = 256
SWITCHING_SIGNAL = 50.0
P_DPE = 0.2
NAN_SAFE_VALUE = 1.0e6

NW = 32
K_MAX = (N_PMTS + NW - 1) // NW
L = 16
STEPS = BATCH // L


def _sc_stage(pred_t, obs_t, coef, lut):
    mesh = plsc.VectorSubcoreMesh(core_axis_name="c", subcore_axis_name="s")

    @functools.partial(
        pl.kernel,
        out_type=jax.ShapeDtypeStruct((NW, BATCH), jnp.float32),
        mesh=mesh,
        scratch_types=[
            pltpu.VMEM((M * M,), jnp.float32),
            pltpu.VMEM((BATCH,), jnp.float32),
            pltpu.VMEM((BATCH,), jnp.float32),
            pltpu.VMEM((BATCH,), jnp.float32),
            pltpu.VMEM((512,), jnp.float32),
        ],
        compiler_params=pltpu.CompilerParams(needs_layout_passes=False),
    )
    def body(pred_hbm, obs_hbm, coef_hbm, lut_hbm, out_hbm,
             tab_v, mu_v, x_v, acc_v, coef_v):
        wid = lax.axis_index("s") * 2 + lax.axis_index("c")

        pltpu.sync_copy(coef_hbm, coef_v)

        @pl.loop(0, STEPS)
        def _zero(i):
            acc_v[pl.ds(i * L, L)] = jnp.zeros((L,), jnp.float32)

        for k in range(K_MAX):
            p = wid + k * NW

            @pl.when(p < N_PMTS)
            def _():
                pltpu.sync_copy(lut_hbm.at[p], tab_v)
                pltpu.sync_copy(pred_hbm.at[p], mu_v)
                pltpu.sync_copy(obs_hbm.at[p], x_v)
                c = coef_v[pl.ds(p, L)][0]

                @pl.loop(0, STEPS)
                def _step(i):
                    base = i * L
                    x = x_v[pl.ds(base, L)]
                    mu = jnp.maximum(mu_v[pl.ds(base, L)], 1e-6)

                    tx = x / SWITCHING_SIGNAL * float(M)
                    tx = jnp.minimum(jnp.maximum(tx, 0.0), float(M - 1))
                    xi = tx.astype(jnp.int32)
                    xi = jnp.minimum(jnp.maximum(xi, 0), M - 1)

                    tm = mu / SWITCHING_SIGNAL * float(M)
                    tm = jnp.minimum(jnp.maximum(tm, 0.0), float(M - 1))
                    mi = tm.astype(jnp.int32)
                    mi = jnp.minimum(jnp.maximum(mi, 0), M - 1)

                    val = plsc.load_gather(tab_v, [xi * M + mi])

                    var = jnp.maximum(mu * c, 1e-12)
                    d = x - mu * (1.0 + P_DPE)
                    g = (d * d) / var

                    use_lut = (x < SWITCHING_SIGNAL) & (mu < SWITCHING_SIGNAL)
                    r = jnp.where(use_lut, val, g)
                    nan = (x != x) | (mu != mu)
                    r = jnp.where(nan, NAN_SAFE_VALUE, r)

                    acc_v[pl.ds(base, L)] = acc_v[pl.ds(base, L)] + r

        pltpu.sync_copy(acc_v, out_hbm.at[wid])

    return body(pred_t, obs_t, coef, lut)


def _tc_reduce(partials):
    def body(p_ref, o_ref):
        o_ref[...] = jnp.sum(p_ref[...], axis=0, keepdims=True)

    return pl.pallas_call(
        body,
        out_shape=jax.ShapeDtypeStruct((1, BATCH), jnp.float32),
    )(partials)


def kernel(pred, observed, std, L_table):
    pred_t = pred.T
    obs_t = observed.T
    lut = L_table.reshape(N_PMTS, M * M)
    coef = (1.0 + P_DPE) ** 2 + P_DPE * (1.0 - P_DPE) + std * std
    coef = jnp.pad(coef, (0, 512 - N_PMTS))
    partials = _sc_stage(pred_t, obs_t, coef, lut)
    return _tc_reduce(partials).reshape(BATCH)
